# baseline (speedup 1.0000x reference)
import functools

import jax
import jax.numpy as jnp
from jax import lax
from jax.experimental import pallas as pl
from jax.experimental.pallas import tpu as pltpu
from jax.experimental.pallas import tpu_sc as plsc

NC = 2
NS = 16
NW = NC * NS
L = 16
CHUNK = 128
LANE = 128
SC_NUM = 3
SC_DEN = 4
TC_BLK = 256
CP_BLK = 512


def _sc_gather(t, table128, b_sc):
    b_per_w = b_sc // NW
    n_chunks = b_per_w // CHUNK
    mesh = plsc.VectorSubcoreMesh(core_axis_name="c", subcore_axis_name="s")

    @functools.partial(
        pl.kernel,
        out_type=jax.ShapeDtypeStruct((b_sc, LANE), jnp.float32),
        mesh=mesh,
        scratch_types=[
            pltpu.VMEM((b_per_w,), jnp.float32),
            pltpu.VMEM((n_chunks, CHUNK), jnp.int32),
            pltpu.VMEM((b_per_w, LANE), jnp.float32),
            pltpu.SemaphoreType.DMA,
            pltpu.SemaphoreType.DMA,
        ],
        compiler_params=pltpu.CompilerParams(use_tc_tiling_on_sc=True),
    )
    def _emb(t_hbm, table_hbm, out_hbm, t_v, idx_v, rows_v, gsem, wsem):
        wid = lax.axis_index("s") * NC + lax.axis_index("c")
        base = wid * b_per_w

        pltpu.sync_copy(t_hbm.at[pl.ds(base, b_per_w)], t_v)

        gathers = []
        for j in range(n_chunks):
            for i in range(CHUNK // L):
                v = t_v[pl.ds(j * CHUNK + i * L, L)]
                idx_v[j, pl.ds(i * L, L)] = (v * 999.0).astype(jnp.int32)
            gathers.append(
                pltpu.async_copy(
                    table_hbm.at[idx_v.at[j]],
                    rows_v.at[pl.ds(j * CHUNK, CHUNK)],
                    gsem,
                )
            )
        writes = []
        for j in range(n_chunks):
            gathers[j].wait()
            writes.append(
                pltpu.async_copy(
                    rows_v.at[pl.ds(j * CHUNK, CHUNK)],
                    out_hbm.at[pl.ds(base + j * CHUNK, CHUNK)],
                    wsem,
                )
            )
        for w in writes:
            w.wait()

    return _emb(t, table128)


def _tc_onehot(t, table, off_blk, b_tc, B, V, D):
    n_blk = b_tc // TC_BLK

    def _body(t_ref, table_ref, out_ref):
        idx = (t_ref[...] * 999.0).astype(jnp.int32)
        iota = lax.broadcasted_iota(jnp.int32, (TC_BLK, V), 1)
        oh = (iota == idx[:, None]).astype(jnp.float32)
        out_ref[...] = jnp.dot(
            oh, table_ref[...], preferred_element_type=jnp.float32
        )

    return pl.pallas_call(
        _body,
        grid=(n_blk,),
        in_specs=[
            pl.BlockSpec((TC_BLK,), lambda i: (i + off_blk,)),
            pl.BlockSpec((V, D), lambda i: (0, 0)),
        ],
        out_specs=pl.BlockSpec((TC_BLK, D), lambda i: (i + off_blk, 0)),
        out_shape=jax.ShapeDtypeStruct((B, D), jnp.float32),
    )(t, table)


def _tc_merge(out_sc, part, b_sc, B, D):
    n_blk = b_sc // CP_BLK

    def _body(sc_ref, part_ref, out_ref):
        out_ref[...] = sc_ref[:, :D]

    return pl.pallas_call(
        _body,
        grid=(n_blk,),
        in_specs=[
            pl.BlockSpec((CP_BLK, LANE), lambda i: (i, 0)),
            pl.BlockSpec(memory_space=pl.ANY),
        ],
        out_specs=pl.BlockSpec((CP_BLK, D), lambda i: (i, 0)),
        out_shape=jax.ShapeDtypeStruct((B, D), jnp.float32),
        input_output_aliases={1: 0},
    )(out_sc, part)


@jax.jit
def kernel(t, table):
    B = t.shape[0]
    V, D = table.shape
    b_sc = B * SC_NUM // SC_DEN
    b_tc = B - b_sc

    table128 = jnp.pad(table, ((0, 0), (0, LANE - D)))
    out_sc = _sc_gather(t, table128, b_sc)
    part = _tc_onehot(t, table, b_sc // TC_BLK, b_tc, B, V, D)
    return _tc_merge(out_sc, part, b_sc, B, D)


# speedup vs baseline: 1.2441x; 1.2441x over previous
import functools

import jax
import jax.numpy as jnp
from jax import lax
from jax.experimental import pallas as pl
from jax.experimental import layout
from jax.experimental.pallas import tpu as pltpu
from jax.experimental.pallas import tpu_sc as plsc

NC = 2
NS = 16
NW = NC * NS
L = 16
CHUNK = 128


def _impl(t, table):
    B = t.shape[0]
    V, D = table.shape
    b_per_w = B // NW
    n_chunks = b_per_w // CHUNK

    mesh = plsc.VectorSubcoreMesh(core_axis_name="c", subcore_axis_name="s")

    @functools.partial(
        pl.kernel,
        out_type=jax.ShapeDtypeStruct((B, D), jnp.float32),
        mesh=mesh,
        scratch_types=[
            pltpu.VMEM((b_per_w,), jnp.float32),
            pltpu.VMEM((n_chunks, CHUNK), jnp.int32),
            pltpu.VMEM((b_per_w, D), jnp.float32),
            pltpu.SemaphoreType.DMA,
            pltpu.SemaphoreType.DMA,
        ],
        compiler_params=pltpu.CompilerParams(use_tc_tiling_on_sc=False),
    )
    def _emb(t_hbm, table_hbm, out_hbm, t_v, idx_v, rows_v, gsem, wsem):
        wid = lax.axis_index("s") * NC + lax.axis_index("c")
        base = wid * b_per_w

        pltpu.sync_copy(t_hbm.at[pl.ds(base, b_per_w)], t_v)

        gathers = []
        for j in range(n_chunks):
            for i in range(CHUNK // L):
                v = t_v[pl.ds(j * CHUNK + i * L, L)]
                idx_v[j, pl.ds(i * L, L)] = (v * 999.0).astype(jnp.int32)
            gathers.append(
                pltpu.async_copy(
                    table_hbm.at[idx_v.at[j]],
                    rows_v.at[pl.ds(j * CHUNK, CHUNK)],
                    gsem,
                )
            )
        writes = []
        for j in range(n_chunks):
            gathers[j].wait()
            writes.append(
                pltpu.async_copy(
                    rows_v.at[pl.ds(j * CHUNK, CHUNK)],
                    out_hbm.at[pl.ds(base + j * CHUNK, CHUNK)],
                    wsem,
                )
            )
        for w in writes:
            w.wait()

    return _emb(t, table)


_impl.__name__ = "kernel"
_JIT_CACHE = {}


def kernel(t, table):
    if isinstance(t, jax.core.Tracer):
        return _impl(t, table)
    try:
        dev = next(iter(t.devices()))
    except Exception:
        dev = jax.devices()[0]
    fn = _JIT_CACHE.get(dev)
    if fn is None:
        fmt = layout.Format(
            layout.Layout(major_to_minor=(0, 1), tiling=()),
            jax.sharding.SingleDeviceSharding(dev),
        )
        fn = jax.jit(_impl, out_shardings=fmt)
        _JIT_CACHE[dev] = fn
    return fn(t, table)
